# baseline (speedup 1.0000x reference)
import functools

import jax
import jax.numpy as jnp
from jax import lax
from jax.experimental import pallas as pl
from jax.experimental.pallas import tpu as pltpu
from jax.experimental.pallas import tpu_sc as plsc

B = 4096
L = 200
LP = 208
D = 64

_BB = 64
_NW = 32
_CHUNK = 256
_NCAND = 16384

_NHIST = B * LP


def _sc_gather_pairs(table_pairs, hist_idx, cand_idx):
    hist_per_w = _NHIST // _NW
    hist_chunks = hist_per_w // _CHUNK
    mesh = plsc.VectorSubcoreMesh(core_axis_name="c", subcore_axis_name="s")

    @functools.partial(
        pl.kernel,
        out_type=(
            jax.ShapeDtypeStruct((_NHIST, 2 * D), table_pairs.dtype),
            jax.ShapeDtypeStruct((_NCAND, 2 * D), table_pairs.dtype),
        ),
        mesh=mesh,
        scratch_types=[
            pltpu.VMEM((_CHUNK,), jnp.int32),
            pltpu.VMEM((_CHUNK,), jnp.int32),
            pltpu.VMEM((_CHUNK, 2 * D), jnp.float32),
            pltpu.VMEM((_CHUNK, 2 * D), jnp.float32),
            pltpu.SemaphoreType.DMA,
            pltpu.SemaphoreType.DMA,
            pltpu.SemaphoreType.DMA,
            pltpu.SemaphoreType.DMA,
            pltpu.SemaphoreType.DMA,
            pltpu.SemaphoreType.DMA,
        ],
    )
    def gather_kernel(tab_hbm, hidx_hbm, cidx_hbm, oh_hbm, oc_hbm,
                      idx_v0, idx_v1, rows_v0, rows_v1,
                      si0, si1, sg0, sg1, sw0, sw1):
        wid = jax.lax.axis_index("s") * 2 + jax.lax.axis_index("c")

        def pair(idx_hbm, out_hbm, b0):
            b1 = b0 + _CHUNK
            c0 = pltpu.async_copy(idx_hbm.at[pl.ds(b0, _CHUNK)], idx_v0, si0)
            c1 = pltpu.async_copy(idx_hbm.at[pl.ds(b1, _CHUNK)], idx_v1, si1)
            c0.wait()
            g0 = pltpu.async_copy(tab_hbm.at[idx_v0], rows_v0, sg0)
            c1.wait()
            g1 = pltpu.async_copy(tab_hbm.at[idx_v1], rows_v1, sg1)
            g0.wait()
            w0 = pltpu.async_copy(rows_v0, out_hbm.at[pl.ds(b0, _CHUNK)], sw0)
            g1.wait()
            w1 = pltpu.async_copy(rows_v1, out_hbm.at[pl.ds(b1, _CHUNK)], sw1)
            w0.wait()
            w1.wait()

        hbase = wid * hist_per_w

        @pl.loop(0, hist_chunks // 2)
        def _(j):
            pair(hidx_hbm, oh_hbm, hbase + 2 * j * _CHUNK)

        pair(cidx_hbm, oc_hbm, wid * 2 * _CHUNK)

    return gather_kernel(table_pairs, hist_idx, cand_idx)


def _att_body(hist_ref, seq_ref, cand_ref, candp_ref, prior_ref, label_ref,
              qw_ref, qb_ref, kwt_ref, kb_ref, vw_ref, vb_ref,
              pw_ref, pb_ref, cw_ref, cb_ref, out_ref):
    seq2 = seq_ref[...]
    h2 = hist_ref[...]
    par3 = (seq2 & 1)[:, :, None]
    emb = jnp.where(par3 == 1, h2[:, :, D:], h2[:, :, :D])

    c2 = cand_ref[...]
    ce = jnp.where(candp_ref[...] == 1, c2[:, D:], c2[:, :D])

    q = jnp.dot(ce, qw_ref[...], preferred_element_type=jnp.float32) + qb_ref[...]
    qp = jnp.dot(q, kwt_ref[...], preferred_element_type=jnp.float32)
    c = jnp.sum(q * kb_ref[...], axis=1, keepdims=True)

    la = lax.dot_general(emb, qp, (((2,), (1,)), ((0,), (0,))),
                         preferred_element_type=jnp.float32)
    la = la + c
    la = jnp.where(seq2 != 0, la, la * (-(2.0 ** 32)))
    lidx = jax.lax.broadcasted_iota(jnp.int32, (_BB, LP), 1)
    la = jnp.where(lidx < L, la, -1e30)

    m = jnp.max(la, axis=1, keepdims=True)
    e = jnp.exp(la - m)
    s = e * (1.0 / jnp.sum(e, axis=1, keepdims=True))

    pooled = lax.dot_general(s, emb, (((1,), (1,)), ((0,), (0,))),
                             preferred_element_type=jnp.float32)
    agg = jnp.dot(pooled, vw_ref[...], preferred_element_type=jnp.float32) + vb_ref[...]
    h = jnp.dot(agg, pw_ref[...], preferred_element_type=jnp.float32) + pb_ref[...]
    lr = jnp.dot(h, cw_ref[...], preferred_element_type=jnp.float32) + cb_ref[...]

    sc = prior_ref[...]
    s0 = (1.0 - sc) * (1.0 - 0.001) + 0.0001
    s1 = sc * (1.0 - 0.001) + 0.0001
    l0 = lr[:, 0:1] + (-jnp.log(1.0 / s0 - 1.0))
    l1 = lr[:, 1:2] + (-jnp.log(1.0 / s1 - 1.0))
    mm = jnp.maximum(l0, l1)
    lse = mm + jnp.log(jnp.exp(l0 - mm) + jnp.exp(l1 - mm))
    lab = label_ref[...].astype(jnp.float32)
    lp_sel = jnp.where(lab > 0.5, l1, l0) - lse
    out_ref[...] = jnp.broadcast_to(-jnp.sum(lp_sel), (1, 1, 1))


def _attention(hist2, seq2, cand2, cand_par, prior_score, label,
               Q_w, Q_b, K_wT, K_b, V_w, V_b, P_w, P_b, C_w, C_b):
    grid = B // _BB
    full = lambda shape: pl.BlockSpec(shape, lambda i: (0,) * len(shape))
    partials = pl.pallas_call(
        _att_body,
        grid=(grid,),
        in_specs=[
            pl.BlockSpec((_BB, LP, 2 * D), lambda i: (i, 0, 0)),
            pl.BlockSpec((_BB, LP), lambda i: (i, 0)),
            pl.BlockSpec((_BB, 2 * D), lambda i: (i, 0)),
            pl.BlockSpec((_BB, 1), lambda i: (i, 0)),
            pl.BlockSpec((_BB, 1), lambda i: (i, 0)),
            pl.BlockSpec((_BB, 1), lambda i: (i, 0)),
            full((D, D)), full((1, D)),
            full((D, D)), full((1, D)),
            full((D, D)), full((1, D)),
            full((D, D)), full((1, D)),
            full((D, 2)), full((1, 2)),
        ],
        out_specs=pl.BlockSpec((1, 1, 1), lambda i: (i, 0, 0)),
        out_shape=jax.ShapeDtypeStruct((grid, 1, 1), jnp.float32),
    )(hist2, seq2, cand2, cand_par, prior_score, label,
      Q_w, Q_b, K_wT, K_b, V_w, V_b, P_w, P_b, C_w, C_b)
    return jnp.sum(partials) / B


def kernel(hist_seq, cand, prior_score, label, emb_table,
           Q_w, Q_b, K_w, K_b, V_w, V_b, P_w, P_b, C_w, C_b):
    v = emb_table.shape[0]
    table_pairs = emb_table.reshape(v // 2, 2 * D)
    seqp = jnp.pad(hist_seq.astype(jnp.int32), ((0, 0), (0, LP - L)),
                   constant_values=2)
    hist_idx = (seqp >> 1).reshape(-1)
    pad_idx = (jnp.arange(_NCAND - B, dtype=jnp.int32) % (v - 1)) + 1
    cand_idx = jnp.concatenate([cand.astype(jnp.int32), pad_idx]) >> 1
    hist_rows, cand_rows = _sc_gather_pairs(table_pairs, hist_idx, cand_idx)
    return _attention(
        hist_rows.reshape(B, LP, 2 * D), seqp,
        cand_rows[:B], (cand & 1).reshape(B, 1),
        prior_score.reshape(B, 1), label.reshape(B, 1).astype(jnp.int32),
        Q_w, Q_b.reshape(1, D), K_w.T, K_b.reshape(1, D),
        V_w, V_b.reshape(1, D), P_w, P_b.reshape(1, D),
        C_w, C_b.reshape(1, 2),
    )


# speedup vs baseline: 1.2014x; 1.2014x over previous
import functools

import jax
import jax.numpy as jnp
from jax import lax
from jax.experimental import pallas as pl
from jax.experimental.pallas import tpu as pltpu
from jax.experimental.pallas import tpu_sc as plsc

B = 4096
L = 200
LP = 208
D = 64

_BB = 64
_NW = 32
_CHUNK = 256
_NCAND = 16384

_NHIST = B * LP


def _sc_gather_pairs(table_pairs, hist_idx, cand_idx):
    hist_per_w = _NHIST // _NW
    hist_chunks = hist_per_w // _CHUNK
    mesh = plsc.VectorSubcoreMesh(core_axis_name="c", subcore_axis_name="s")

    @functools.partial(
        pl.kernel,
        out_type=(
            jax.ShapeDtypeStruct((_NHIST, 2 * D), table_pairs.dtype),
            jax.ShapeDtypeStruct((_NCAND, 2 * D), table_pairs.dtype),
        ),
        mesh=mesh,
        scratch_types=[
            pltpu.VMEM((_CHUNK,), jnp.int32),
            pltpu.VMEM((_CHUNK,), jnp.int32),
            pltpu.VMEM((_CHUNK, 2 * D), jnp.float32),
            pltpu.VMEM((_CHUNK, 2 * D), jnp.float32),
            pltpu.SemaphoreType.DMA,
            pltpu.SemaphoreType.DMA,
            pltpu.SemaphoreType.DMA,
            pltpu.SemaphoreType.DMA,
            pltpu.SemaphoreType.DMA,
            pltpu.SemaphoreType.DMA,
        ],
    )
    def gather_kernel(tab_hbm, hidx_hbm, cidx_hbm, oh_hbm, oc_hbm,
                      idx_v0, idx_v1, rows_v0, rows_v1,
                      si0, si1, sg0, sg1, sw0, sw1):
        wid = jax.lax.axis_index("s") * 2 + jax.lax.axis_index("c")

        def pair(idx_hbm, out_hbm, b0):
            b1 = b0 + _CHUNK
            c0 = pltpu.async_copy(idx_hbm.at[pl.ds(b0, _CHUNK)], idx_v0, si0)
            c1 = pltpu.async_copy(idx_hbm.at[pl.ds(b1, _CHUNK)], idx_v1, si1)
            c0.wait()
            g0 = pltpu.async_copy(tab_hbm.at[idx_v0], rows_v0, sg0)
            c1.wait()
            g1 = pltpu.async_copy(tab_hbm.at[idx_v1], rows_v1, sg1)
            g0.wait()
            w0 = pltpu.async_copy(rows_v0, out_hbm.at[pl.ds(b0, _CHUNK)], sw0)
            g1.wait()
            w1 = pltpu.async_copy(rows_v1, out_hbm.at[pl.ds(b1, _CHUNK)], sw1)
            w0.wait()
            w1.wait()

        hbase = wid * hist_per_w

        @pl.loop(0, hist_chunks // 2)
        def _(j):
            pair(hidx_hbm, oh_hbm, hbase + 2 * j * _CHUNK)

        pair(cidx_hbm, oc_hbm, wid * 2 * _CHUNK)

    return gather_kernel(table_pairs, hist_idx, cand_idx)


def _att_body(hist_ref, seq_ref, cand_ref, candp_ref, prior_ref, label_ref,
              qw_ref, qb_ref, kwt_ref, kb_ref, vw_ref, vb_ref,
              pw_ref, pb_ref, cw_ref, cb_ref, out_ref):
    seq3 = seq_ref[...][:, :, None]
    h2 = hist_ref[...]
    emb = jnp.where((seq3 & 1) == 1, h2[:, :, D:], h2[:, :, :D])

    c2 = cand_ref[...]
    ce = jnp.where(candp_ref[...] == 1, c2[:, D:], c2[:, :D])

    q = jnp.dot(ce, qw_ref[...], preferred_element_type=jnp.float32) + qb_ref[...]
    qp = jnp.dot(q, kwt_ref[...], preferred_element_type=jnp.float32)
    c = jnp.sum(q * kb_ref[...], axis=1, keepdims=True)

    prod = emb * qp[:, None, :]
    ones = jnp.full((D, D), 1.0, jnp.float32)
    la = jnp.dot(prod.reshape(_BB * LP, D), ones,
                 preferred_element_type=jnp.float32).reshape(_BB, LP, D)
    la = la + c[:, :, None]
    la = jnp.where(seq3 != 0, la, la * (-(2.0 ** 32)))
    lidx = jax.lax.broadcasted_iota(jnp.int32, (_BB, LP, 1), 1)
    la = jnp.where(lidx < L, la, -1e30)

    m = jnp.max(la, axis=1, keepdims=True)
    e = jnp.exp(la - m)
    s = e * (1.0 / jnp.sum(e, axis=1, keepdims=True))

    pooled = jnp.sum(s * emb, axis=1)
    agg = jnp.dot(pooled, vw_ref[...], preferred_element_type=jnp.float32) + vb_ref[...]
    h = jnp.dot(agg, pw_ref[...], preferred_element_type=jnp.float32) + pb_ref[...]
    lr = jnp.dot(h, cw_ref[...], preferred_element_type=jnp.float32) + cb_ref[...]

    sc = prior_ref[...]
    s0 = (1.0 - sc) * (1.0 - 0.001) + 0.0001
    s1 = sc * (1.0 - 0.001) + 0.0001
    l0 = lr[:, 0:1] + (-jnp.log(1.0 / s0 - 1.0))
    l1 = lr[:, 1:2] + (-jnp.log(1.0 / s1 - 1.0))
    mm = jnp.maximum(l0, l1)
    lse = mm + jnp.log(jnp.exp(l0 - mm) + jnp.exp(l1 - mm))
    lab = label_ref[...].astype(jnp.float32)
    lp_sel = jnp.where(lab > 0.5, l1, l0) - lse
    out_ref[...] = jnp.broadcast_to(-jnp.sum(lp_sel), (1, 1, 1))


def _attention(hist2, seq2, cand2, cand_par, prior_score, label,
               Q_w, Q_b, K_wT, K_b, V_w, V_b, P_w, P_b, C_w, C_b):
    grid = B // _BB
    full = lambda shape: pl.BlockSpec(shape, lambda i: (0,) * len(shape))
    partials = pl.pallas_call(
        _att_body,
        grid=(grid,),
        in_specs=[
            pl.BlockSpec((_BB, LP, 2 * D), lambda i: (i, 0, 0)),
            pl.BlockSpec((_BB, LP), lambda i: (i, 0)),
            pl.BlockSpec((_BB, 2 * D), lambda i: (i, 0)),
            pl.BlockSpec((_BB, 1), lambda i: (i, 0)),
            pl.BlockSpec((_BB, 1), lambda i: (i, 0)),
            pl.BlockSpec((_BB, 1), lambda i: (i, 0)),
            full((D, D)), full((1, D)),
            full((D, D)), full((1, D)),
            full((D, D)), full((1, D)),
            full((D, D)), full((1, D)),
            full((D, 2)), full((1, 2)),
        ],
        out_specs=pl.BlockSpec((1, 1, 1), lambda i: (i, 0, 0)),
        out_shape=jax.ShapeDtypeStruct((grid, 1, 1), jnp.float32),
    )(hist2, seq2, cand2, cand_par, prior_score, label,
      Q_w, Q_b, K_wT, K_b, V_w, V_b, P_w, P_b, C_w, C_b)
    return jnp.sum(partials) / B


def kernel(hist_seq, cand, prior_score, label, emb_table,
           Q_w, Q_b, K_w, K_b, V_w, V_b, P_w, P_b, C_w, C_b):
    v = emb_table.shape[0]
    table_pairs = emb_table.reshape(v // 2, 2 * D)
    seqp = jnp.pad(hist_seq.astype(jnp.int32), ((0, 0), (0, LP - L)),
                   constant_values=2)
    hist_idx = (seqp >> 1).reshape(-1)
    pad_idx = (jnp.arange(_NCAND - B, dtype=jnp.int32) % (v - 1)) + 1
    cand_idx = jnp.concatenate([cand.astype(jnp.int32), pad_idx]) >> 1
    hist_rows, cand_rows = _sc_gather_pairs(table_pairs, hist_idx, cand_idx)
    return _attention(
        hist_rows.reshape(B, LP, 2 * D), seqp,
        cand_rows[:B], (cand & 1).reshape(B, 1),
        prior_score.reshape(B, 1), label.reshape(B, 1).astype(jnp.int32),
        Q_w, Q_b.reshape(1, D), K_w.T, K_b.reshape(1, D),
        V_w, V_b.reshape(1, D), P_w, P_b.reshape(1, D),
        C_w, C_b.reshape(1, 2),
    )


# speedup vs baseline: 1.9936x; 1.6594x over previous
import functools

import jax
import jax.numpy as jnp
from jax import lax
from jax.experimental import pallas as pl
from jax.experimental.pallas import tpu as pltpu
from jax.experimental.pallas import tpu_sc as plsc

B = 4096
L = 200
LP = 208
D = 64

_BB = 64
_NW = 32
_CHUNK = 256
_NCAND = 16384

_NHIST = B * LP


def _sc_gather_pairs(table_pairs, hist_idx, cand_idx):
    mesh = plsc.VectorSubcoreMesh(core_axis_name="c", subcore_axis_name="s")

    @functools.partial(
        pl.kernel,
        out_type=(
            jax.ShapeDtypeStruct((_NHIST, 2 * D), table_pairs.dtype),
            jax.ShapeDtypeStruct((_NCAND, 2 * D), table_pairs.dtype),
        ),
        mesh=mesh,
        scratch_types=[
            pltpu.VMEM((_CHUNK,), jnp.int32),
            pltpu.VMEM((_CHUNK,), jnp.int32),
            pltpu.VMEM((_CHUNK, 2 * D), jnp.float32),
            pltpu.VMEM((_CHUNK, 2 * D), jnp.float32),
            pltpu.SemaphoreType.DMA,
            pltpu.SemaphoreType.DMA,
            pltpu.SemaphoreType.DMA,
            pltpu.SemaphoreType.DMA,
            pltpu.SemaphoreType.DMA,
            pltpu.SemaphoreType.DMA,
        ],
    )
    def gather_kernel(tab_hbm, hidx_hbm, cidx_hbm, oh_hbm, oc_hbm,
                      idx_v0, idx_v1, rows_v0, rows_v1,
                      si0, si1, sg0, sg1, sw0, sw1):
        wid = jax.lax.axis_index("s") * 2 + jax.lax.axis_index("c")

        def pair(n, i0, i1, o0, o1):
            c0 = pltpu.async_copy(hidx_hbm.at[pl.ds(i0, n)] if n == L
                                  else cidx_hbm.at[pl.ds(i0, n)],
                                  idx_v0.at[pl.ds(0, n)], si0)
            c1 = pltpu.async_copy(hidx_hbm.at[pl.ds(i1, n)] if n == L
                                  else cidx_hbm.at[pl.ds(i1, n)],
                                  idx_v1.at[pl.ds(0, n)], si1)
            c0.wait()
            g0 = pltpu.async_copy(tab_hbm.at[idx_v0.at[pl.ds(0, n)]],
                                  rows_v0.at[pl.ds(0, n)], sg0)
            c1.wait()
            g1 = pltpu.async_copy(tab_hbm.at[idx_v1.at[pl.ds(0, n)]],
                                  rows_v1.at[pl.ds(0, n)], sg1)
            g0.wait()
            w0 = pltpu.async_copy(rows_v0.at[pl.ds(0, n)],
                                  (oh_hbm if n == L else oc_hbm).at[pl.ds(o0, n)],
                                  sw0)
            g1.wait()
            w1 = pltpu.async_copy(rows_v1.at[pl.ds(0, n)],
                                  (oh_hbm if n == L else oc_hbm).at[pl.ds(o1, n)],
                                  sw1)
            w0.wait()
            w1.wait()

        wb = wid * (B // _NW)

        @pl.loop(0, B // _NW // 2)
        def _(j):
            b = wb + 2 * j
            pair(L, b * L, (b + 1) * L, b * LP, (b + 1) * LP)

        pair(_CHUNK, wid * 2 * _CHUNK, wid * 2 * _CHUNK + _CHUNK,
             wid * 2 * _CHUNK, wid * 2 * _CHUNK + _CHUNK)

    return gather_kernel(table_pairs, hist_idx, cand_idx)


def _att_body(hist_ref, seq_ref, cand_ref, candp_ref, prior_ref, label_ref,
              qw_ref, qb_ref, kwt_ref, kb_ref, vw_ref, vb_ref,
              pw_ref, pb_ref, cw_ref, cb_ref, out_ref):
    seq3 = seq_ref[...]
    h2 = hist_ref[...]
    lidx3 = jax.lax.broadcasted_iota(jnp.int32, (_BB, LP, 1), 1)
    emb = jnp.where((seq3 & 1) == 1, h2[:, :, D:], h2[:, :, :D])
    emb = jnp.where(lidx3 < L, emb, 0.0)

    c2 = cand_ref[...]
    ce = jnp.where(candp_ref[...] == 1, c2[:, D:], c2[:, :D])

    q = jnp.dot(ce, qw_ref[...], preferred_element_type=jnp.float32) + qb_ref[...]
    qp = jnp.dot(q, kwt_ref[...], preferred_element_type=jnp.float32)
    c = jnp.sum(q * kb_ref[...], axis=1, keepdims=True)

    prod = emb * qp[:, None, :]
    ones = jnp.full((D, D), 1.0, jnp.float32)
    la = jnp.dot(prod.reshape(_BB * LP, D), ones,
                 preferred_element_type=jnp.float32).reshape(_BB, LP, D)
    la = la + c[:, :, None]
    la = jnp.where(seq3 != 0, la, la * (-(2.0 ** 32)))
    la = jnp.where(lidx3 < L, la, -1e30)

    m = jnp.max(la, axis=1, keepdims=True)
    e = jnp.exp(la - m)
    s = e * (1.0 / jnp.sum(e, axis=1, keepdims=True))

    pooled = jnp.sum(s * emb, axis=1)
    agg = jnp.dot(pooled, vw_ref[...], preferred_element_type=jnp.float32) + vb_ref[...]
    h = jnp.dot(agg, pw_ref[...], preferred_element_type=jnp.float32) + pb_ref[...]
    lr = jnp.dot(h, cw_ref[...], preferred_element_type=jnp.float32) + cb_ref[...]

    sc = prior_ref[...]
    s0 = (1.0 - sc) * (1.0 - 0.001) + 0.0001
    s1 = sc * (1.0 - 0.001) + 0.0001
    l0 = lr[:, 0:1] + (-jnp.log(1.0 / s0 - 1.0))
    l1 = lr[:, 1:2] + (-jnp.log(1.0 / s1 - 1.0))
    mm = jnp.maximum(l0, l1)
    lse = mm + jnp.log(jnp.exp(l0 - mm) + jnp.exp(l1 - mm))
    lab = label_ref[...].astype(jnp.float32)
    lp_sel = jnp.where(lab > 0.5, l1, l0) - lse
    out_ref[...] = jnp.broadcast_to(-jnp.sum(lp_sel), (1, 1, 1))


def _attention(hist2, seq2, cand2, cand_par, prior_score, label,
               Q_w, Q_b, K_wT, K_b, V_w, V_b, P_w, P_b, C_w, C_b):
    grid = B // _BB
    full = lambda shape: pl.BlockSpec(shape, lambda i: (0,) * len(shape))
    partials = pl.pallas_call(
        _att_body,
        grid=(grid,),
        in_specs=[
            pl.BlockSpec((_BB, LP, 2 * D), lambda i: (i, 0, 0)),
            pl.BlockSpec((_BB, LP, 1), lambda i: (i, 0, 0)),
            pl.BlockSpec((_BB, 2 * D), lambda i: (i, 0)),
            pl.BlockSpec((_BB, 1), lambda i: (i, 0)),
            pl.BlockSpec((_BB, 1), lambda i: (i, 0)),
            pl.BlockSpec((_BB, 1), lambda i: (i, 0)),
            full((D, D)), full((1, D)),
            full((D, D)), full((1, D)),
            full((D, D)), full((1, D)),
            full((D, D)), full((1, D)),
            full((D, 2)), full((1, 2)),
        ],
        out_specs=pl.BlockSpec((1, 1, 1), lambda i: (i, 0, 0)),
        out_shape=jax.ShapeDtypeStruct((grid, 1, 1), jnp.float32),
    )(hist2, seq2, cand2, cand_par, prior_score, label,
      Q_w, Q_b, K_wT, K_b, V_w, V_b, P_w, P_b, C_w, C_b)
    return jnp.sum(partials) / B


def kernel(hist_seq, cand, prior_score, label, emb_table,
           Q_w, Q_b, K_w, K_b, V_w, V_b, P_w, P_b, C_w, C_b):
    v = emb_table.shape[0]
    table_pairs = emb_table.reshape(v // 2, 2 * D)
    seqp3 = jnp.pad(hist_seq.astype(jnp.int32).reshape(B, L, 1),
                    ((0, 0), (0, LP - L), (0, 0)), constant_values=2)
    hist_idx = (hist_seq.astype(jnp.int32) >> 1).reshape(-1)
    pad_idx = (jnp.arange(_NCAND - B, dtype=jnp.int32) % (v - 1)) + 1
    cand_idx = jnp.concatenate([cand.astype(jnp.int32), pad_idx]) >> 1
    hist_rows, cand_rows = _sc_gather_pairs(table_pairs, hist_idx, cand_idx)
    return _attention(
        hist_rows.reshape(B, LP, 2 * D), seqp3,
        cand_rows[:B], (cand & 1).reshape(B, 1),
        prior_score.reshape(B, 1), label.reshape(B, 1).astype(jnp.int32),
        Q_w, Q_b.reshape(1, D), K_w.T, K_b.reshape(1, D),
        V_w, V_b.reshape(1, D), P_w, P_b.reshape(1, D),
        C_w, C_b.reshape(1, 2),
    )


# speedup vs baseline: 2.2889x; 1.1481x over previous
import functools

import jax
import jax.numpy as jnp
from jax.experimental import pallas as pl
from jax.experimental.pallas import tpu as pltpu
from jax.experimental.pallas import tpu_sc as plsc

B = 4096
L = 200
D = 64

_BB = 64
_NW = 32
_CHUNK = 256
_NCAND = 16384


def _sc_gather_pairs(table_pairs, hist_idx, cand_idx):
    n_hist = hist_idx.shape[0]
    hist_per_w = n_hist // _NW
    hist_chunks = hist_per_w // _CHUNK
    mesh = plsc.VectorSubcoreMesh(core_axis_name="c", subcore_axis_name="s")

    @functools.partial(
        pl.kernel,
        out_type=(
            jax.ShapeDtypeStruct((n_hist, 2 * D), table_pairs.dtype),
            jax.ShapeDtypeStruct((_NCAND, 2 * D), table_pairs.dtype),
        ),
        mesh=mesh,
        scratch_types=[
            pltpu.VMEM((_CHUNK,), jnp.int32),
            pltpu.VMEM((_CHUNK,), jnp.int32),
            pltpu.VMEM((_CHUNK, 2 * D), jnp.float32),
            pltpu.VMEM((_CHUNK, 2 * D), jnp.float32),
            pltpu.SemaphoreType.DMA,
            pltpu.SemaphoreType.DMA,
            pltpu.SemaphoreType.DMA,
            pltpu.SemaphoreType.DMA,
            pltpu.SemaphoreType.DMA,
            pltpu.SemaphoreType.DMA,
        ],
    )
    def gather_kernel(tab_hbm, hidx_hbm, cidx_hbm, oh_hbm, oc_hbm,
                      idx_v0, idx_v1, rows_v0, rows_v1,
                      si0, si1, sg0, sg1, sw0, sw1):
        wid = jax.lax.axis_index("s") * 2 + jax.lax.axis_index("c")

        def pair(idx_hbm, out_hbm, b0):
            b1 = b0 + _CHUNK
            c0 = pltpu.async_copy(idx_hbm.at[pl.ds(b0, _CHUNK)], idx_v0, si0)
            c1 = pltpu.async_copy(idx_hbm.at[pl.ds(b1, _CHUNK)], idx_v1, si1)
            c0.wait()
            g0 = pltpu.async_copy(tab_hbm.at[idx_v0], rows_v0, sg0)
            c1.wait()
            g1 = pltpu.async_copy(tab_hbm.at[idx_v1], rows_v1, sg1)
            g0.wait()
            w0 = pltpu.async_copy(rows_v0, out_hbm.at[pl.ds(b0, _CHUNK)], sw0)
            g1.wait()
            w1 = pltpu.async_copy(rows_v1, out_hbm.at[pl.ds(b1, _CHUNK)], sw1)
            w0.wait()
            w1.wait()

        hbase = wid * hist_per_w

        @pl.loop(0, hist_chunks // 2)
        def _(j):
            pair(hidx_hbm, oh_hbm, hbase + 2 * j * _CHUNK)

        pair(cidx_hbm, oc_hbm, wid * 2 * _CHUNK)

    return gather_kernel(table_pairs, hist_idx, cand_idx)


def _att_body(hist_ref, seq_ref, cand_ref, candp_ref, prior_ref, label_ref,
              qw_ref, qb_ref, kwt_ref, kb_ref, vw_ref, vb_ref,
              pw_ref, pb_ref, cw_ref, cb_ref, out_ref):
    seq3 = seq_ref[...]
    h2 = hist_ref[...]
    emb = jnp.where((seq3 & 1) == 1, h2[:, :, D:], h2[:, :, :D])

    c2 = cand_ref[...]
    ce = jnp.where(candp_ref[...] == 1, c2[:, D:], c2[:, :D])

    q = jnp.dot(ce, qw_ref[...], preferred_element_type=jnp.float32) + qb_ref[...]
    qp = jnp.dot(q, kwt_ref[...], preferred_element_type=jnp.float32)
    c = jnp.sum(q * kb_ref[...], axis=1, keepdims=True)

    prod = emb * qp[:, None, :]
    ones = jnp.full((D, D), 1.0, jnp.float32)
    la = jnp.dot(prod.reshape(_BB * L, D), ones,
                 preferred_element_type=jnp.float32).reshape(_BB, L, D)
    la = la + c[:, :, None]
    la = jnp.where(seq3 != 0, la, la * (-(2.0 ** 32)))

    m = jnp.max(la, axis=1, keepdims=True)
    e = jnp.exp(la - m)
    s = e * (1.0 / jnp.sum(e, axis=1, keepdims=True))

    pooled = jnp.sum(s * emb, axis=1)
    agg = jnp.dot(pooled, vw_ref[...], preferred_element_type=jnp.float32) + vb_ref[...]
    h = jnp.dot(agg, pw_ref[...], preferred_element_type=jnp.float32) + pb_ref[...]
    lr = jnp.dot(h, cw_ref[...], preferred_element_type=jnp.float32) + cb_ref[...]

    sc = prior_ref[...]
    s0 = (1.0 - sc) * (1.0 - 0.001) + 0.0001
    s1 = sc * (1.0 - 0.001) + 0.0001
    l0 = lr[:, 0:1] + (-jnp.log(1.0 / s0 - 1.0))
    l1 = lr[:, 1:2] + (-jnp.log(1.0 / s1 - 1.0))
    mm = jnp.maximum(l0, l1)
    lse = mm + jnp.log(jnp.exp(l0 - mm) + jnp.exp(l1 - mm))
    lab = label_ref[...].astype(jnp.float32)
    lp_sel = jnp.where(lab > 0.5, l1, l0) - lse
    out_ref[...] = jnp.broadcast_to(-jnp.sum(lp_sel), (1, 1, 1))


def _attention(hist2, seq3, cand2, cand_par, prior_score, label,
               Q_w, Q_b, K_wT, K_b, V_w, V_b, P_w, P_b, C_w, C_b):
    grid = B // _BB
    full = lambda shape: pl.BlockSpec(shape, lambda i: (0,) * len(shape))
    partials = pl.pallas_call(
        _att_body,
        grid=(grid,),
        in_specs=[
            pl.BlockSpec((_BB, L, 2 * D), lambda i: (i, 0, 0)),
            pl.BlockSpec((_BB, L, 1), lambda i: (i, 0, 0)),
            pl.BlockSpec((_BB, 2 * D), lambda i: (i, 0)),
            pl.BlockSpec((_BB, 1), lambda i: (i, 0)),
            pl.BlockSpec((_BB, 1), lambda i: (i, 0)),
            pl.BlockSpec((_BB, 1), lambda i: (i, 0)),
            full((D, D)), full((1, D)),
            full((D, D)), full((1, D)),
            full((D, D)), full((1, D)),
            full((D, D)), full((1, D)),
            full((D, 2)), full((1, 2)),
        ],
        out_specs=pl.BlockSpec((1, 1, 1), lambda i: (i, 0, 0)),
        out_shape=jax.ShapeDtypeStruct((grid, 1, 1), jnp.float32),
    )(hist2, seq3, cand2, cand_par, prior_score, label,
      Q_w, Q_b, K_wT, K_b, V_w, V_b, P_w, P_b, C_w, C_b)
    return jnp.sum(partials) / B


def kernel(hist_seq, cand, prior_score, label, emb_table,
           Q_w, Q_b, K_w, K_b, V_w, V_b, P_w, P_b, C_w, C_b):
    v = emb_table.shape[0]
    table_pairs = emb_table.reshape(v // 2, 2 * D)
    pad_idx = (jnp.arange(_NCAND - B, dtype=jnp.int32) % (v - 1)) + 1
    hist_idx = hist_seq.reshape(-1).astype(jnp.int32) >> 1
    cand_idx = jnp.concatenate([cand.astype(jnp.int32), pad_idx]) >> 1
    hist_rows, cand_rows = _sc_gather_pairs(table_pairs, hist_idx, cand_idx)
    hist2 = hist_rows.reshape(B, L, 2 * D)
    cand2 = cand_rows[:B]
    return _attention(
        hist2, hist_seq.reshape(B, L, 1), cand2, (cand & 1).reshape(B, 1),
        prior_score.reshape(B, 1), label.reshape(B, 1).astype(jnp.int32),
        Q_w, Q_b.reshape(1, D), K_w.T, K_b.reshape(1, D),
        V_w, V_b.reshape(1, D), P_w, P_b.reshape(1, D),
        C_w, C_b.reshape(1, 2),
    )
